# Initial kernel scaffold; baseline (speedup 1.0000x reference)
import jax
import jax.numpy as jnp
from jax.experimental import pallas as pl


def kernel(points):
    raise NotImplementedError("write your pallas kernel here")


# speedup vs baseline: 1.8501x; 1.8501x over previous
import functools

import jax
import jax.numpy as jnp
import numpy as np
from jax import lax
from jax.experimental import pallas as pl
from jax.experimental.pallas import tpu as pltpu
from jax.experimental.pallas import tpu_sc as plsc

N = 1_200_000
NS = 1_204_224
PADN = NS - N
MAXP = 10
GX = (2000, 1000, 500)
GY = (2000, 1000, 500)
GZ = (40, 20, 10)
VSX = (0.1, 0.2, 0.4)
VSZ = (0.2, 0.4, 0.8)
NVOX = (60000, 40000, 20000)
OFF = (0, 160_000_000, 180_000_000)
SENT = 182_500_000

VOXB = (0, 600_000, 1_000_000)
VOXDUMP = 1_200_000
VOXSZ = 1_212_416
NPB = (0, 60_160, 100_480)
NPDUMP = 120_960
NPSZ = 131_072
DUMP_MASK = 8191

W = 128
PER_TILE = NS // 16
NWIN = PER_TILE // W
MCHUNK = 2048
NVCH = VOXSZ // (16 * MCHUNK)
NPCH = NPSZ // (16 * MCHUNK)


def _key_body(x_ref, y_ref, z_ref, key_ref, inv_ref):
    x = x_ref[0, 0, :]
    y = y_ref[0, 0, :]
    z = z_ref[0, 0, :]
    d = jnp.sqrt(x * x + y * y)
    in0 = d < np.float32(30.0)
    in1 = d < np.float32(60.0)
    in2 = d < np.float32(100.1)
    inzone = (in0, (~in0) & in1, (~in1) & in2)
    key = jnp.full(x.shape, SENT, jnp.int32)
    inv = jnp.zeros(x.shape, jnp.int32)
    for zi in range(3):
        cx = jnp.floor((x + np.float32(100.0)) / np.float32(VSX[zi])).astype(jnp.int32)
        cy = jnp.floor((y + np.float32(100.0)) / np.float32(VSX[zi])).astype(jnp.int32)
        cz = jnp.floor((z + np.float32(5.0)) / np.float32(VSZ[zi])).astype(jnp.int32)
        ingrid = ((cx >= 0) & (cx < GX[zi]) & (cy >= 0) & (cy < GY[zi])
                  & (cz >= 0) & (cz < GZ[zi]))
        vz = inzone[zi] & ingrid
        lk = (cz * GY[zi] + cy) * GX[zi] + cx
        key = jnp.where(vz, OFF[zi] + lk, key)
        inv = inv | jnp.where(vz, 0, 1 << zi)
    key_ref[0, 0, :] = key
    inv_ref[0, 0, :] = inv


def _compute_keys(pts_pad):
    nb = NS // 4096
    xs = [pts_pad[:, i].reshape(nb, 1, 4096) for i in range(3)]
    spec = pl.BlockSpec((1, 1, 4096), lambda i: (i, 0, 0))
    keys, inv = pl.pallas_call(
        _key_body,
        grid=(nb,),
        in_specs=[spec, spec, spec],
        out_specs=[spec, spec],
        out_shape=[jax.ShapeDtypeStruct((nb, 1, 4096), jnp.int32)] * 2,
    )(*xs)
    return keys.reshape(NS), inv.reshape(NS)


_MESH = plsc.VectorSubcoreMesh(core_axis_name="c", subcore_axis_name="s")


@functools.partial(
    pl.kernel,
    mesh=_MESH,
    compiler_params=pltpu.CompilerParams(use_tc_tiling_on_sc=False),
    out_type=(
        [jax.ShapeDtypeStruct((VOXSZ,), jnp.float32)] * 4
        + [jax.ShapeDtypeStruct((NPSZ,), jnp.int32)] * 4
    ),
    scratch_types=[
        pltpu.VMEM((W,), jnp.int32),
        pltpu.VMEM((W,), jnp.float32),
        pltpu.VMEM((W,), jnp.int32),
        pltpu.VMEM((W,), jnp.int32),
        pltpu.VMEM((W,), jnp.int32),
        pltpu.VMEM((MCHUNK,), jnp.float32),
        pltpu.VMEM((MCHUNK,), jnp.int32),
        pltpu.SemaphoreType.DMA,
    ],
)
def _sc_scatter(px_hbm, py_hbm, pz_hbm, pi_hbm, sidx_hbm, didxv_hbm,
                didxnp_hbm, valnp_hbm, didxvc_hbm, vcz_hbm, vcy_hbm, vcx_hbm,
                zvox_hbm, znp_hbm, fz_hbm, fy_hbm, fx_hbm,
                vox0_out, vox1_out, vox2_out, vox3_out,
                np_out, vcz_out, vcy_out, vcx_out,
                gsrc_v, dat_v, dvi_v, dni_v, val_v, zv_b, zn_b, sem):
    c = lax.axis_index("c")
    s = lax.axis_index("s")
    vox_outs = (vox0_out, vox1_out, vox2_out, vox3_out)
    pcomp = (px_hbm, py_hbm, pz_hbm, pi_hbm)
    vc_outs = (vcz_out, vcy_out, vcx_out)
    vc_srcs = (vcz_hbm, vcy_hbm, vcx_hbm)
    vc_fills = (fz_hbm, fy_hbm, fx_hbm)

    @pl.when(c == 0)
    def _core0():
        pltpu.sync_copy(zvox_hbm, zv_b)

        def mz(k, carry):
            dst = pl.ds((s * NVCH + k) * MCHUNK, MCHUNK)
            for j in range(4):
                pltpu.sync_copy(zv_b, vox_outs[j].at[dst])
            return carry
        lax.fori_loop(0, NVCH, mz, 0)
        plsc.subcore_barrier()

        base = s * PER_TILE

        def win(k, carry):
            st = base + k * W
            pltpu.sync_copy(sidx_hbm.at[pl.ds(st, W)], gsrc_v)
            pltpu.sync_copy(didxv_hbm.at[pl.ds(st, W)], dvi_v)
            for j in range(4):
                pltpu.async_copy(pcomp[j].at[gsrc_v], dat_v, sem).wait()
                pltpu.async_copy(dat_v, vox_outs[j].at[dvi_v], sem).wait()
            return carry
        lax.fori_loop(0, NWIN, win, 0)

    @pl.when(c == 1)
    def _core1():
        pltpu.sync_copy(znp_hbm, zn_b)

        def mz(k, carry):
            ch = (s * NPCH + k) * MCHUNK
            dst = pl.ds(ch, MCHUNK)
            pltpu.sync_copy(zn_b, np_out.at[dst])
            for j in range(3):
                pltpu.sync_copy(vc_fills[j].at[dst], zn_b)
                pltpu.sync_copy(zn_b, vc_outs[j].at[dst])
            pltpu.sync_copy(znp_hbm, zn_b)
            return carry
        lax.fori_loop(0, NPCH, mz, 0)
        plsc.subcore_barrier()

        base = s * PER_TILE

        def win(k, carry):
            st = base + k * W
            pltpu.sync_copy(didxnp_hbm.at[pl.ds(st, W)], dni_v)
            pltpu.sync_copy(valnp_hbm.at[pl.ds(st, W)], val_v)
            pltpu.async_copy(val_v, np_out.at[dni_v], sem).wait()
            pltpu.sync_copy(didxvc_hbm.at[pl.ds(st, W)], dni_v)
            for j in range(3):
                pltpu.sync_copy(vc_srcs[j].at[pl.ds(st, W)], val_v)
                pltpu.async_copy(val_v, vc_outs[j].at[dni_v], sem).wait()
            return carry
        lax.fori_loop(0, NWIN, win, 0)


def kernel(points):
    pts_pad = jnp.concatenate(
        [points, jnp.full((PADN, 4), 1e9, jnp.float32)], axis=0)
    keys, invb = _compute_keys(pts_pad)

    iota = jnp.arange(NS, dtype=jnp.int32)
    skey, sidx = lax.sort((keys, iota), num_keys=1, is_stable=True)

    flag = jnp.concatenate([jnp.ones((1,), bool), skey[1:] != skey[:-1]])
    last = jnp.concatenate([skey[:-1] != skey[1:], jnp.ones((1,), bool)])
    uid = jnp.cumsum(flag.astype(jnp.int32)) - 1
    run_start = lax.cummax(jnp.where(flag, iota, 0))
    pos = iota - run_start
    run_len = pos + 1
    valid = skey < SENT
    zs = (skey >= OFF[1]).astype(jnp.int32) + (skey >= OFF[2]).astype(jnp.int32)
    u1 = jnp.sum((flag & (skey < OFF[1])).astype(jnp.int32))
    u2 = jnp.sum((flag & (skey < OFF[2])).astype(jnp.int32))
    ubase = jnp.stack([jnp.zeros((), jnp.int32), u1, u2])[zs]
    sinv = uid - ubase
    nvox_s = jnp.asarray(NVOX, jnp.int32)[zs]
    keep = valid & (pos < MAXP) & (sinv < nvox_s)

    dump = iota & DUMP_MASK
    voxb_s = jnp.asarray(VOXB, jnp.int32)[zs]
    didx_v = jnp.where(keep, voxb_s + sinv * MAXP + pos, VOXDUMP + dump)
    npb_s = jnp.asarray(NPB, jnp.int32)[zs]
    lmask = valid & last & (sinv < nvox_s)
    didx_np = jnp.where(lmask, npb_s + sinv, NPDUMP + dump)
    val_np = jnp.minimum(run_len, MAXP)
    fmask = valid & flag & (sinv < nvox_s)
    didx_vc = jnp.where(fmask, npb_s + sinv, NPDUMP + dump)
    offv = jnp.asarray(OFF, jnp.int32)[zs]
    gxv = jnp.asarray(GX, jnp.int32)[zs]
    gyv = jnp.asarray(GY, jnp.int32)[zs]
    lk = skey - offv
    cx = lk % gxv
    cyq = lk // gxv
    cy = cyq % gyv
    cz = cyq // gyv

    iota_n = iota[:N]
    fills = []
    for zi in range(3):
        bit = (invb[:N] >> zi) & 1
        istar = jnp.max(jnp.where(bit == 1, iota_n, -1))
        istar = jnp.maximum(istar, 0)
        p = points[istar]
        fx = jnp.floor((p[0] + np.float32(100.0)) / np.float32(VSX[zi])).astype(jnp.int32)
        fy = jnp.floor((p[1] + np.float32(100.0)) / np.float32(VSX[zi])).astype(jnp.int32)
        fz = jnp.floor((p[2] + np.float32(5.0)) / np.float32(VSZ[zi])).astype(jnp.int32)
        fills.append((fz, fy, fx))
    fill_planes = []
    for j in range(3):
        fill_planes.append(jnp.concatenate([
            jnp.full((NPB[1] - NPB[0],), fills[0][j], jnp.int32),
            jnp.full((NPB[2] - NPB[1],), fills[1][j], jnp.int32),
            jnp.full((NPDUMP - NPB[2],), fills[2][j], jnp.int32),
            jnp.zeros((NPSZ - NPDUMP,), jnp.int32),
        ]))
    zvox = jnp.zeros((MCHUNK,), jnp.float32)
    znp = jnp.zeros((MCHUNK,), jnp.int32)

    res = _sc_scatter(
        pts_pad[:, 0], pts_pad[:, 1], pts_pad[:, 2], pts_pad[:, 3],
        sidx, didx_v, didx_np, val_np, didx_vc, cz, cy, cx,
        zvox, znp, fill_planes[0], fill_planes[1], fill_planes[2])
    vox_planes = res[:4]
    npts_flat = res[4]
    vc_planes = res[5:8]

    vox_flat = jnp.stack(vox_planes, axis=1)
    vc_flat = jnp.stack(vc_planes, axis=1)
    outs = []
    for zi in range(3):
        v = vox_flat[VOXB[zi]:VOXB[zi] + NVOX[zi] * MAXP].reshape(
            NVOX[zi], MAXP, 4)
        vc = vc_flat[NPB[zi]:NPB[zi] + NVOX[zi]]
        npts = npts_flat[NPB[zi]:NPB[zi] + NVOX[zi]]
        outs.extend([v, vc, npts])
    return tuple(outs)


# speedup vs baseline: 1.9346x; 1.0457x over previous
import functools

import jax
import jax.numpy as jnp
import numpy as np
from jax import lax
from jax.experimental import pallas as pl
from jax.experimental.pallas import tpu as pltpu
from jax.experimental.pallas import tpu_sc as plsc

N = 1_200_000
NS = 1_204_224
PADN = NS - N
MAXP = 10
GX = (2000, 1000, 500)
GY = (2000, 1000, 500)
GZ = (40, 20, 10)
VSX = (0.1, 0.2, 0.4)
VSZ = (0.2, 0.4, 0.8)
NVOX = (60000, 40000, 20000)
OFF = (0, 160_000_000, 180_000_000)
SENT = 182_500_000

VOXB = (0, 600_000, 1_000_000)
VOXDUMP = 1_200_000
VOXSZ = 1_212_416
NPB = (0, 60_160, 100_480)
NPDUMP = 120_960
NPSZ = 131_072
DUMP_MASK = 8191

W = 512
PER_TILE = NS // 16
NWIN = PER_TILE // W
NPAIR = NWIN // 2
MCHUNK = 2048
NVCH = VOXSZ // (16 * MCHUNK)
NPCH = NPSZ // (16 * MCHUNK)


def _key_body(x_ref, y_ref, z_ref, key_ref, inv_ref):
    x = x_ref[0, 0, :]
    y = y_ref[0, 0, :]
    z = z_ref[0, 0, :]
    d = jnp.sqrt(x * x + y * y)
    in0 = d < np.float32(30.0)
    in1 = d < np.float32(60.0)
    in2 = d < np.float32(100.1)
    inzone = (in0, (~in0) & in1, (~in1) & in2)
    key = jnp.full(x.shape, SENT, jnp.int32)
    inv = jnp.zeros(x.shape, jnp.int32)
    for zi in range(3):
        cx = jnp.floor((x + np.float32(100.0)) / np.float32(VSX[zi])).astype(jnp.int32)
        cy = jnp.floor((y + np.float32(100.0)) / np.float32(VSX[zi])).astype(jnp.int32)
        cz = jnp.floor((z + np.float32(5.0)) / np.float32(VSZ[zi])).astype(jnp.int32)
        ingrid = ((cx >= 0) & (cx < GX[zi]) & (cy >= 0) & (cy < GY[zi])
                  & (cz >= 0) & (cz < GZ[zi]))
        vz = inzone[zi] & ingrid
        lk = (cz * GY[zi] + cy) * GX[zi] + cx
        key = jnp.where(vz, OFF[zi] + lk, key)
        inv = inv | jnp.where(vz, 0, 1 << zi)
    key_ref[0, 0, :] = key
    inv_ref[0, 0, :] = inv


def _compute_keys(pts_pad):
    nb = NS // 4096
    xs = [pts_pad[:, i].reshape(nb, 1, 4096) for i in range(3)]
    spec = pl.BlockSpec((1, 1, 4096), lambda i: (i, 0, 0))
    keys, inv = pl.pallas_call(
        _key_body,
        grid=(nb,),
        in_specs=[spec, spec, spec],
        out_specs=[spec, spec],
        out_shape=[jax.ShapeDtypeStruct((nb, 1, 4096), jnp.int32)] * 2,
    )(*xs)
    return keys.reshape(NS), inv.reshape(NS)


_MESH = plsc.VectorSubcoreMesh(core_axis_name="c", subcore_axis_name="s")


@functools.partial(
    pl.kernel,
    mesh=_MESH,
    compiler_params=pltpu.CompilerParams(use_tc_tiling_on_sc=False),
    out_type=(
        [jax.ShapeDtypeStruct((VOXSZ,), jnp.float32)] * 4
        + [jax.ShapeDtypeStruct((NPSZ,), jnp.int32)] * 4
    ),
    scratch_types=(
        [pltpu.VMEM((W,), jnp.int32)] * 16
        + [pltpu.VMEM((W,), jnp.float32)] * 8
        + [pltpu.VMEM((MCHUNK,), jnp.float32),
           pltpu.VMEM((MCHUNK,), jnp.int32)]
        + [pltpu.SemaphoreType.DMA] * 5
    ),
)
def _sc_scatter(px_hbm, py_hbm, pz_hbm, pi_hbm, sidx_hbm, didxv_hbm,
                didxnp_hbm, valnp_hbm, didxvc_hbm, vcz_hbm, vcy_hbm, vcx_hbm,
                zvox_hbm, znp_hbm, fz_hbm, fy_hbm, fx_hbm,
                vox0_out, vox1_out, vox2_out, vox3_out,
                np_out, vcz_out, vcy_out, vcx_out,
                i0, i1, i2, i3, i4, i5, i6, i7,
                i8, i9, i10, i11, i12, i13, i14, i15,
                f0, f1, f2, f3, f4, f5, f6, f7,
                zv_b, zn_b, semm, semg0, semg1, sems0, sems1):
    c = lax.axis_index("c")
    s = lax.axis_index("s")
    vox_outs = (vox0_out, vox1_out, vox2_out, vox3_out)
    pcomp = (px_hbm, py_hbm, pz_hbm, pi_hbm)
    vc_outs = (vcz_out, vcy_out, vcx_out)
    vc_srcs = (vcz_hbm, vcy_hbm, vcx_hbm)
    vc_fills = (fz_hbm, fy_hbm, fx_hbm)
    datA = (f0, f1, f2, f3)
    datB = (f4, f5, f6, f7)

    @pl.when(c == 0)
    def _core0():
        pltpu.sync_copy(zvox_hbm, zv_b)

        def mz(k, carry):
            dst = pl.ds((s * NVCH + k) * MCHUNK, MCHUNK)
            hs = [pltpu.async_copy(zv_b, vox_outs[j].at[dst], semm)
                  for j in range(4)]
            for h in hs:
                h.wait()
            return carry
        lax.fori_loop(0, NVCH, mz, 0)
        plsc.subcore_barrier()

        base = s * PER_TILE
        sA, dA, sB, dB = i0, i1, i2, i3

        def gswin(st, sbuf, dbuf, dat, semg, sems):
            hg = [pltpu.async_copy(pcomp[j].at[sbuf], dat[j], semg)
                  for j in range(4)]
            return hg

        def pair(k, carry):
            st0 = base + (2 * k) * W
            st1 = st0 + W
            pltpu.sync_copy(sidx_hbm.at[pl.ds(st0, W)], sA)
            pltpu.sync_copy(didxv_hbm.at[pl.ds(st0, W)], dA)
            hg0 = gswin(st0, sA, dA, datA, semg0, sems0)
            pltpu.sync_copy(sidx_hbm.at[pl.ds(st1, W)], sB)
            pltpu.sync_copy(didxv_hbm.at[pl.ds(st1, W)], dB)
            for h in hg0:
                h.wait()
            hs0 = [pltpu.async_copy(datA[j], vox_outs[j].at[dA], sems0)
                   for j in range(4)]
            hg1 = gswin(st1, sB, dB, datB, semg1, sems1)
            for h in hg1:
                h.wait()
            hs1 = [pltpu.async_copy(datB[j], vox_outs[j].at[dB], sems1)
                   for j in range(4)]
            for h in hs0:
                h.wait()
            for h in hs1:
                h.wait()
            return carry
        lax.fori_loop(0, NPAIR, pair, 0)

        st = base + (2 * NPAIR) * W
        pltpu.sync_copy(sidx_hbm.at[pl.ds(st, W)], sA)
        pltpu.sync_copy(didxv_hbm.at[pl.ds(st, W)], dA)
        hg = [pltpu.async_copy(pcomp[j].at[sA], datA[j], semg0)
              for j in range(4)]
        for h in hg:
            h.wait()
        hs = [pltpu.async_copy(datA[j], vox_outs[j].at[dA], sems0)
              for j in range(4)]
        for h in hs:
            h.wait()

    @pl.when(c == 1)
    def _core1():
        pltpu.sync_copy(znp_hbm, zn_b)

        def mz(k, carry):
            ch = (s * NPCH + k) * MCHUNK
            dst = pl.ds(ch, MCHUNK)
            pltpu.sync_copy(zn_b, np_out.at[dst])
            for j in range(3):
                pltpu.sync_copy(vc_fills[j].at[dst], zn_b)
                pltpu.sync_copy(zn_b, vc_outs[j].at[dst])
            pltpu.sync_copy(znp_hbm, zn_b)
            return carry
        lax.fori_loop(0, NPCH, mz, 0)
        plsc.subcore_barrier()

        base = s * PER_TILE
        dnA, vnA, dcA, czA, cyA, cxA = i4, i5, i6, i7, i8, i9
        dnB, vnB, dcB, czB, cyB, cxB = i10, i11, i12, i13, i14, i15

        def load(st, dn, vn, dc, cz_, cy_, cx_):
            pltpu.sync_copy(didxnp_hbm.at[pl.ds(st, W)], dn)
            pltpu.sync_copy(valnp_hbm.at[pl.ds(st, W)], vn)
            pltpu.sync_copy(didxvc_hbm.at[pl.ds(st, W)], dc)
            for j, b in enumerate((cz_, cy_, cx_)):
                pltpu.sync_copy(vc_srcs[j].at[pl.ds(st, W)], b)

        def fire(dn, vn, dc, cz_, cy_, cx_, sem):
            hs = [pltpu.async_copy(vn, np_out.at[dn], sem)]
            for j, b in enumerate((cz_, cy_, cx_)):
                hs.append(pltpu.async_copy(b, vc_outs[j].at[dc], sem))
            return hs

        def pair(k, carry):
            st0 = base + (2 * k) * W
            st1 = st0 + W
            load(st0, dnA, vnA, dcA, czA, cyA, cxA)
            hs0 = fire(dnA, vnA, dcA, czA, cyA, cxA, sems0)
            load(st1, dnB, vnB, dcB, czB, cyB, cxB)
            hs1 = fire(dnB, vnB, dcB, czB, cyB, cxB, sems1)
            for h in hs0:
                h.wait()
            for h in hs1:
                h.wait()
            return carry
        lax.fori_loop(0, NPAIR, pair, 0)

        st = base + (2 * NPAIR) * W
        load(st, dnA, vnA, dcA, czA, cyA, cxA)
        hs = fire(dnA, vnA, dcA, czA, cyA, cxA, sems0)
        for h in hs:
            h.wait()


def kernel(points):
    pts_pad = jnp.concatenate(
        [points, jnp.full((PADN, 4), 1e9, jnp.float32)], axis=0)
    keys, invb = _compute_keys(pts_pad)

    iota = jnp.arange(NS, dtype=jnp.int32)
    skey, sidx = lax.sort((keys, iota), num_keys=1, is_stable=True)

    flag = jnp.concatenate([jnp.ones((1,), bool), skey[1:] != skey[:-1]])
    last = jnp.concatenate([skey[:-1] != skey[1:], jnp.ones((1,), bool)])
    uid = jnp.cumsum(flag.astype(jnp.int32)) - 1
    run_start = lax.cummax(jnp.where(flag, iota, 0))
    pos = iota - run_start
    run_len = pos + 1
    valid = skey < SENT
    zs = (skey >= OFF[1]).astype(jnp.int32) + (skey >= OFF[2]).astype(jnp.int32)
    u1 = jnp.sum((flag & (skey < OFF[1])).astype(jnp.int32))
    u2 = jnp.sum((flag & (skey < OFF[2])).astype(jnp.int32))
    ubase = jnp.stack([jnp.zeros((), jnp.int32), u1, u2])[zs]
    sinv = uid - ubase
    nvox_s = jnp.asarray(NVOX, jnp.int32)[zs]
    keep = valid & (pos < MAXP) & (sinv < nvox_s)

    dump = iota & DUMP_MASK
    voxb_s = jnp.asarray(VOXB, jnp.int32)[zs]
    didx_v = jnp.where(keep, voxb_s + sinv * MAXP + pos, VOXDUMP + dump)
    npb_s = jnp.asarray(NPB, jnp.int32)[zs]
    lmask = valid & last & (sinv < nvox_s)
    didx_np = jnp.where(lmask, npb_s + sinv, NPDUMP + dump)
    val_np = jnp.minimum(run_len, MAXP)
    fmask = valid & flag & (sinv < nvox_s)
    didx_vc = jnp.where(fmask, npb_s + sinv, NPDUMP + dump)
    offv = jnp.asarray(OFF, jnp.int32)[zs]
    gxv = jnp.asarray(GX, jnp.int32)[zs]
    gyv = jnp.asarray(GY, jnp.int32)[zs]
    lk = skey - offv
    cx = lk % gxv
    cyq = lk // gxv
    cy = cyq % gyv
    cz = cyq // gyv

    iota_n = iota[:N]
    fills = []
    for zi in range(3):
        bit = (invb[:N] >> zi) & 1
        istar = jnp.max(jnp.where(bit == 1, iota_n, -1))
        istar = jnp.maximum(istar, 0)
        p = points[istar]
        fx = jnp.floor((p[0] + np.float32(100.0)) / np.float32(VSX[zi])).astype(jnp.int32)
        fy = jnp.floor((p[1] + np.float32(100.0)) / np.float32(VSX[zi])).astype(jnp.int32)
        fz = jnp.floor((p[2] + np.float32(5.0)) / np.float32(VSZ[zi])).astype(jnp.int32)
        fills.append((fz, fy, fx))
    fill_planes = []
    for j in range(3):
        fill_planes.append(jnp.concatenate([
            jnp.full((NPB[1] - NPB[0],), fills[0][j], jnp.int32),
            jnp.full((NPB[2] - NPB[1],), fills[1][j], jnp.int32),
            jnp.full((NPDUMP - NPB[2],), fills[2][j], jnp.int32),
            jnp.zeros((NPSZ - NPDUMP,), jnp.int32),
        ]))
    zvox = jnp.zeros((MCHUNK,), jnp.float32)
    znp = jnp.zeros((MCHUNK,), jnp.int32)

    res = _sc_scatter(
        pts_pad[:, 0], pts_pad[:, 1], pts_pad[:, 2], pts_pad[:, 3],
        sidx, didx_v, didx_np, val_np, didx_vc, cz, cy, cx,
        zvox, znp, fill_planes[0], fill_planes[1], fill_planes[2])
    vox_planes = res[:4]
    npts_flat = res[4]
    vc_planes = res[5:8]

    vox_flat = jnp.stack(vox_planes, axis=1)
    vc_flat = jnp.stack(vc_planes, axis=1)
    outs = []
    for zi in range(3):
        v = vox_flat[VOXB[zi]:VOXB[zi] + NVOX[zi] * MAXP].reshape(
            NVOX[zi], MAXP, 4)
        vc = vc_flat[NPB[zi]:NPB[zi] + NVOX[zi]]
        npts = npts_flat[NPB[zi]:NPB[zi] + NVOX[zi]]
        outs.extend([v, vc, npts])
    return tuple(outs)


# speedup vs baseline: 2.2317x; 1.1536x over previous
import functools

import jax
import jax.numpy as jnp
import numpy as np
from jax import lax
from jax.experimental import pallas as pl
from jax.experimental.pallas import tpu as pltpu
from jax.experimental.pallas import tpu_sc as plsc

N = 1_200_000
NS = 1_204_224
PADN = NS - N
MAXP = 10
GX = (2000, 1000, 500)
GY = (2000, 1000, 500)
GZ = (40, 20, 10)
VSX = (0.1, 0.2, 0.4)
VSZ = (0.2, 0.4, 0.8)
NVOX = (60000, 40000, 20000)
OFF = (0, 160_000_000, 180_000_000)
SENT = 182_500_000

VOXB = (0, 600_000, 1_000_000)
VOXZERO = 1_212_416
VOXSZ = VOXZERO + NS
NPB = (0, 60_160, 100_480)
NPZERO = 131_072
NPSZ = NPZERO + NS

W = 512
PER_TILE = NS // 16
NWIN = PER_TILE // W
NPAIR = NWIN // 2
MCHUNK = 2048
NVCH = VOXZERO // (16 * MCHUNK)
NPCH = NPZERO // (16 * MCHUNK)


def _key_body(x_ref, y_ref, z_ref, key_ref, inv_ref):
    x = x_ref[0, 0, :]
    y = y_ref[0, 0, :]
    z = z_ref[0, 0, :]
    d = jnp.sqrt(x * x + y * y)
    in0 = d < np.float32(30.0)
    in1 = d < np.float32(60.0)
    in2 = d < np.float32(100.1)
    inzone = (in0, (~in0) & in1, (~in1) & in2)
    key = jnp.full(x.shape, SENT, jnp.int32)
    inv = jnp.zeros(x.shape, jnp.int32)
    for zi in range(3):
        cx = jnp.floor((x + np.float32(100.0)) / np.float32(VSX[zi])).astype(jnp.int32)
        cy = jnp.floor((y + np.float32(100.0)) / np.float32(VSX[zi])).astype(jnp.int32)
        cz = jnp.floor((z + np.float32(5.0)) / np.float32(VSZ[zi])).astype(jnp.int32)
        ingrid = ((cx >= 0) & (cx < GX[zi]) & (cy >= 0) & (cy < GY[zi])
                  & (cz >= 0) & (cz < GZ[zi]))
        vz = inzone[zi] & ingrid
        lk = (cz * GY[zi] + cy) * GX[zi] + cx
        key = jnp.where(vz, OFF[zi] + lk, key)
        inv = inv | jnp.where(vz, 0, 1 << zi)
    key_ref[0, 0, :] = key
    inv_ref[0, 0, :] = inv


def _compute_keys(pts_pad):
    nb = NS // 4096
    xs = [pts_pad[:, i].reshape(nb, 1, 4096) for i in range(3)]
    spec = pl.BlockSpec((1, 1, 4096), lambda i: (i, 0, 0))
    keys, inv = pl.pallas_call(
        _key_body,
        grid=(nb,),
        in_specs=[spec, spec, spec],
        out_specs=[spec, spec],
        out_shape=[jax.ShapeDtypeStruct((nb, 1, 4096), jnp.int32)] * 2,
    )(*xs)
    return keys.reshape(NS), inv.reshape(NS)


_MESH = plsc.VectorSubcoreMesh(core_axis_name="c", subcore_axis_name="s")


@functools.partial(
    pl.kernel,
    mesh=_MESH,
    compiler_params=pltpu.CompilerParams(use_tc_tiling_on_sc=False),
    out_type=(
        [jax.ShapeDtypeStruct((VOXSZ,), jnp.float32)] * 4
        + [jax.ShapeDtypeStruct((NPSZ,), jnp.int32)] * 4
    ),
    scratch_types=(
        [pltpu.VMEM((W,), jnp.int32)] * 16
        + [pltpu.VMEM((W,), jnp.float32)] * 8
        + [pltpu.VMEM((MCHUNK,), jnp.float32),
           pltpu.VMEM((MCHUNK,), jnp.int32)]
        + [pltpu.SemaphoreType.DMA] * 5
    ),
)
def _sc_scatter(px_hbm, py_hbm, pz_hbm, pi_hbm, sidx_hbm, didxv_hbm,
                didxnp_hbm, valnp_hbm, didxvc_hbm, vcz_hbm, vcy_hbm, vcx_hbm,
                zvox_hbm, znp_hbm, fz_hbm, fy_hbm, fx_hbm,
                vox0_out, vox1_out, vox2_out, vox3_out,
                np_out, vcz_out, vcy_out, vcx_out,
                i0, i1, i2, i3, i4, i5, i6, i7,
                i8, i9, i10, i11, i12, i13, i14, i15,
                f0, f1, f2, f3, f4, f5, f6, f7,
                zv_b, zn_b, semm, semg0, semg1, sems0, sems1):
    c = lax.axis_index("c")
    s = lax.axis_index("s")
    vox_outs = (vox0_out, vox1_out, vox2_out, vox3_out)
    pcomp = (px_hbm, py_hbm, pz_hbm, pi_hbm)
    vc_outs = (vcz_out, vcy_out, vcx_out)
    vc_srcs = (vcz_hbm, vcy_hbm, vcx_hbm)
    vc_fills = (fz_hbm, fy_hbm, fx_hbm)
    datA = (f0, f1, f2, f3)
    datB = (f4, f5, f6, f7)

    @pl.when(c == 0)
    def _core0():
        pltpu.sync_copy(zvox_hbm, zv_b)

        def mz(k, carry):
            dst = pl.ds((s * NVCH + k) * MCHUNK, MCHUNK)
            hs = [pltpu.async_copy(zv_b, vox_outs[j].at[dst], semm)
                  for j in range(4)]
            for h in hs:
                h.wait()
            return carry
        lax.fori_loop(0, NVCH, mz, 0)
        plsc.subcore_barrier()

        base = s * PER_TILE
        sA, dA, sB, dB = i0, i1, i2, i3

        def gswin(st, sbuf, dbuf, dat, semg, sems):
            hg = [pltpu.async_copy(pcomp[j].at[sbuf], dat[j], semg)
                  for j in range(4)]
            return hg

        def pair(k, carry):
            st0 = base + (2 * k) * W
            st1 = st0 + W
            pltpu.sync_copy(sidx_hbm.at[pl.ds(st0, W)], sA)
            pltpu.sync_copy(didxv_hbm.at[pl.ds(st0, W)], dA)
            hg0 = gswin(st0, sA, dA, datA, semg0, sems0)
            pltpu.sync_copy(sidx_hbm.at[pl.ds(st1, W)], sB)
            pltpu.sync_copy(didxv_hbm.at[pl.ds(st1, W)], dB)
            for h in hg0:
                h.wait()
            hs0 = [pltpu.async_copy(datA[j], vox_outs[j].at[dA], sems0)
                   for j in range(4)]
            hg1 = gswin(st1, sB, dB, datB, semg1, sems1)
            for h in hg1:
                h.wait()
            hs1 = [pltpu.async_copy(datB[j], vox_outs[j].at[dB], sems1)
                   for j in range(4)]
            for h in hs0:
                h.wait()
            for h in hs1:
                h.wait()
            return carry
        lax.fori_loop(0, NPAIR, pair, 0)

        st = base + (2 * NPAIR) * W
        pltpu.sync_copy(sidx_hbm.at[pl.ds(st, W)], sA)
        pltpu.sync_copy(didxv_hbm.at[pl.ds(st, W)], dA)
        hg = [pltpu.async_copy(pcomp[j].at[sA], datA[j], semg0)
              for j in range(4)]
        for h in hg:
            h.wait()
        hs = [pltpu.async_copy(datA[j], vox_outs[j].at[dA], sems0)
              for j in range(4)]
        for h in hs:
            h.wait()

    @pl.when(c == 1)
    def _core1():
        pltpu.sync_copy(znp_hbm, zn_b)

        def mz(k, carry):
            ch = (s * NPCH + k) * MCHUNK
            dst = pl.ds(ch, MCHUNK)
            pltpu.sync_copy(zn_b, np_out.at[dst])
            for j in range(3):
                pltpu.sync_copy(vc_fills[j].at[dst], zn_b)
                pltpu.sync_copy(zn_b, vc_outs[j].at[dst])
            pltpu.sync_copy(znp_hbm, zn_b)
            return carry
        lax.fori_loop(0, NPCH, mz, 0)
        plsc.subcore_barrier()

        base = s * PER_TILE
        dnA, vnA, dcA, czA, cyA, cxA = i4, i5, i6, i7, i8, i9
        dnB, vnB, dcB, czB, cyB, cxB = i10, i11, i12, i13, i14, i15

        def load(st, dn, vn, dc, cz_, cy_, cx_):
            pltpu.sync_copy(didxnp_hbm.at[pl.ds(st, W)], dn)
            pltpu.sync_copy(valnp_hbm.at[pl.ds(st, W)], vn)
            pltpu.sync_copy(didxvc_hbm.at[pl.ds(st, W)], dc)
            for j, b in enumerate((cz_, cy_, cx_)):
                pltpu.sync_copy(vc_srcs[j].at[pl.ds(st, W)], b)

        def fire(dn, vn, dc, cz_, cy_, cx_, sem):
            hs = [pltpu.async_copy(vn, np_out.at[dn], sem)]
            for j, b in enumerate((cz_, cy_, cx_)):
                hs.append(pltpu.async_copy(b, vc_outs[j].at[dc], sem))
            return hs

        def pair(k, carry):
            st0 = base + (2 * k) * W
            st1 = st0 + W
            load(st0, dnA, vnA, dcA, czA, cyA, cxA)
            hs0 = fire(dnA, vnA, dcA, czA, cyA, cxA, sems0)
            load(st1, dnB, vnB, dcB, czB, cyB, cxB)
            hs1 = fire(dnB, vnB, dcB, czB, cyB, cxB, sems1)
            for h in hs0:
                h.wait()
            for h in hs1:
                h.wait()
            return carry
        lax.fori_loop(0, NPAIR, pair, 0)

        st = base + (2 * NPAIR) * W
        load(st, dnA, vnA, dcA, czA, cyA, cxA)
        hs = fire(dnA, vnA, dcA, czA, cyA, cxA, sems0)
        for h in hs:
            h.wait()


def kernel(points):
    pts_pad = jnp.concatenate(
        [points, jnp.full((PADN, 4), 1e9, jnp.float32)], axis=0)
    keys, invb = _compute_keys(pts_pad)

    iota = jnp.arange(NS, dtype=jnp.int32)
    skey, sidx = lax.sort((keys, iota), num_keys=1, is_stable=True)

    flag = jnp.concatenate([jnp.ones((1,), bool), skey[1:] != skey[:-1]])
    last = jnp.concatenate([skey[:-1] != skey[1:], jnp.ones((1,), bool)])
    uid = jnp.cumsum(flag.astype(jnp.int32)) - 1
    run_start = lax.cummax(jnp.where(flag, iota, 0))
    pos = iota - run_start
    run_len = pos + 1
    valid = skey < SENT
    zs = (skey >= OFF[1]).astype(jnp.int32) + (skey >= OFF[2]).astype(jnp.int32)
    u1 = jnp.sum((flag & (skey < OFF[1])).astype(jnp.int32))
    u2 = jnp.sum((flag & (skey < OFF[2])).astype(jnp.int32))
    ubase = jnp.stack([jnp.zeros((), jnp.int32), u1, u2])[zs]
    sinv = uid - ubase
    nvox_s = jnp.asarray(NVOX, jnp.int32)[zs]
    keep = valid & (pos < MAXP) & (sinv < nvox_s)

    voxb_s = jnp.asarray(VOXB, jnp.int32)[zs]
    didx_v = jnp.where(keep, voxb_s + sinv * MAXP + pos, VOXZERO + iota)
    npb_s = jnp.asarray(NPB, jnp.int32)[zs]
    lmask = valid & last & (sinv < nvox_s)
    didx_np = jnp.where(lmask, npb_s + sinv, NPZERO + iota)
    val_np = jnp.minimum(run_len, MAXP)
    fmask = valid & flag & (sinv < nvox_s)
    didx_vc = jnp.where(fmask, npb_s + sinv, NPZERO + iota)
    offv = jnp.asarray(OFF, jnp.int32)[zs]
    gxv = jnp.asarray(GX, jnp.int32)[zs]
    gyv = jnp.asarray(GY, jnp.int32)[zs]
    lk = skey - offv
    cx = lk % gxv
    cyq = lk // gxv
    cy = cyq % gyv
    cz = cyq // gyv

    iota_n = iota[:N]
    fills = []
    for zi in range(3):
        bit = (invb[:N] >> zi) & 1
        istar = jnp.max(jnp.where(bit == 1, iota_n, -1))
        istar = jnp.maximum(istar, 0)
        p = points[istar]
        fx = jnp.floor((p[0] + np.float32(100.0)) / np.float32(VSX[zi])).astype(jnp.int32)
        fy = jnp.floor((p[1] + np.float32(100.0)) / np.float32(VSX[zi])).astype(jnp.int32)
        fz = jnp.floor((p[2] + np.float32(5.0)) / np.float32(VSZ[zi])).astype(jnp.int32)
        fills.append((fz, fy, fx))
    fill_planes = []
    for j in range(3):
        fill_planes.append(jnp.concatenate([
            jnp.full((NPB[1] - NPB[0],), fills[0][j], jnp.int32),
            jnp.full((NPB[2] - NPB[1],), fills[1][j], jnp.int32),
            jnp.full((120_960 - NPB[2],), fills[2][j], jnp.int32),
            jnp.zeros((NPZERO - 120_960,), jnp.int32),
        ]))
    zvox = jnp.zeros((MCHUNK,), jnp.float32)
    znp = jnp.zeros((MCHUNK,), jnp.int32)

    res = _sc_scatter(
        pts_pad[:, 0], pts_pad[:, 1], pts_pad[:, 2], pts_pad[:, 3],
        sidx, didx_v, didx_np, val_np, didx_vc, cz, cy, cx,
        zvox, znp, fill_planes[0], fill_planes[1], fill_planes[2])
    vox_planes = res[:4]
    npts_flat = res[4]
    vc_planes = res[5:8]

    vox_flat = jnp.stack(vox_planes, axis=1)
    vc_flat = jnp.stack(vc_planes, axis=1)
    outs = []
    for zi in range(3):
        v = vox_flat[VOXB[zi]:VOXB[zi] + NVOX[zi] * MAXP].reshape(
            NVOX[zi], MAXP, 4)
        vc = vc_flat[NPB[zi]:NPB[zi] + NVOX[zi]]
        npts = npts_flat[NPB[zi]:NPB[zi] + NVOX[zi]]
        outs.extend([v, vc, npts])
    return tuple(outs)


# speedup vs baseline: 9.9989x; 4.4803x over previous
import functools

import jax
import jax.numpy as jnp
import numpy as np
from jax import lax
from jax.experimental import pallas as pl
from jax.experimental.pallas import tpu as pltpu
from jax.experimental.pallas import tpu_sc as plsc

N = 1_200_000
NS = 1_204_224
PADN = NS - N
MAXP = 10
GX = (2000, 1000, 500)
GY = (2000, 1000, 500)
GZ = (40, 20, 10)
VSX = (0.1, 0.2, 0.4)
VSZ = (0.2, 0.4, 0.8)
NVOX = (60000, 40000, 20000)
OFF = (0, 160_000_000, 180_000_000)
SENT = 182_500_000

VOXB = (0, 600_000, 1_000_000)
VOXTOT = 1_200_000
NPB = (0, 60_160, 100_480)
NPTOT = 120_960

CH = 2048
NCHV = (VOXTOT + CH - 1) // CH
NCHN = (NPTOT + CH - 1) // CH
VOXSZ = NCHV * CH
NPSZ = NCHN * CH
PIECE_V = 1024
PIECE_N = 2048
PBV_PAD = 592
PBN_PAD = 64
VOX_W = 16
NP_W = 5
VC_W = 11
VOX_ROUNDS = 37
NP_ROUNDS = 12
VC_ROUNDS = 6
DUMPV = 2_000_000
DUMPN = 1_000_000


def _key_body(x_ref, y_ref, z_ref, key_ref, inv_ref):
    x = x_ref[0, 0, :]
    y = y_ref[0, 0, :]
    z = z_ref[0, 0, :]
    d = jnp.sqrt(x * x + y * y)
    in0 = d < np.float32(30.0)
    in1 = d < np.float32(60.0)
    in2 = d < np.float32(100.1)
    inzone = (in0, (~in0) & in1, (~in1) & in2)
    key = jnp.full(x.shape, SENT, jnp.int32)
    inv = jnp.zeros(x.shape, jnp.int32)
    for zi in range(3):
        cx = jnp.floor((x + np.float32(100.0)) / np.float32(VSX[zi])).astype(jnp.int32)
        cy = jnp.floor((y + np.float32(100.0)) / np.float32(VSX[zi])).astype(jnp.int32)
        cz = jnp.floor((z + np.float32(5.0)) / np.float32(VSZ[zi])).astype(jnp.int32)
        ingrid = ((cx >= 0) & (cx < GX[zi]) & (cy >= 0) & (cy < GY[zi])
                  & (cz >= 0) & (cz < GZ[zi]))
        vz = inzone[zi] & ingrid
        lk = (cz * GY[zi] + cy) * GX[zi] + cx
        key = jnp.where(vz, OFF[zi] + lk, key)
        inv = inv | jnp.where(vz, 0, 1 << zi)
    key_ref[0, 0, :] = key
    inv_ref[0, 0, :] = inv


def _compute_keys(pts_pad):
    nb = NS // 4096
    xs = [pts_pad[:, i].reshape(nb, 1, 4096) for i in range(3)]
    spec = pl.BlockSpec((1, 1, 4096), lambda i: (i, 0, 0))
    keys, inv = pl.pallas_call(
        _key_body,
        grid=(nb,),
        in_specs=[spec, spec, spec],
        out_specs=[spec, spec],
        out_shape=[jax.ShapeDtypeStruct((nb, 1, 4096), jnp.int32)] * 2,
    )(*xs)
    return keys.reshape(NS), inv.reshape(NS)


_MESH = plsc.VectorSubcoreMesh(core_axis_name="c", subcore_axis_name="s")


def _pbval(pb_vm, idx):
    base = (idx // 16) * 16
    v = pb_vm[pl.ds(base, 16)]
    lane = lax.broadcasted_iota(jnp.int32, (16,), 0)
    sel = jnp.where(lane == (idx - base), v, 0)
    return jnp.max(sel)


def _span(pb_vm, c):
    p0 = (_pbval(pb_vm, c) // 16) * 16
    p1 = _pbval(pb_vm, c + 1)
    return p0, p1


@functools.partial(
    pl.kernel,
    mesh=_MESH,
    compiler_params=pltpu.CompilerParams(
        use_tc_tiling_on_sc=False, needs_layout_passes=False),
    out_type=(
        [jax.ShapeDtypeStruct((VOXSZ,), jnp.float32)] * 4
        + [jax.ShapeDtypeStruct((NPSZ,), jnp.int32)] * 4
    ),
    scratch_types=(
        [pltpu.VMEM((PBV_PAD,), jnp.int32),
         pltpu.VMEM((PBN_PAD,), jnp.int32),
         pltpu.VMEM((PBN_PAD,), jnp.int32)]
        + [pltpu.VMEM((PIECE_N,), jnp.int32)] * 4
        + [pltpu.VMEM((PIECE_V,), jnp.float32)] * 4
        + [pltpu.VMEM((CH,), jnp.float32)] * 4
        + [pltpu.VMEM((CH,), jnp.int32)] * 4
        + [pltpu.SemaphoreType.DMA]
    ),
)
def _sc_bin(didxv_hbm, sp0_hbm, sp1_hbm, sp2_hbm, sp3_hbm,
            didxn_hbm, valn_hbm, didxc_hbm, vcz_hbm, vcy_hbm, vcx_hbm,
            pbv_hbm, pbn_hbm, pbc_hbm,
            vox0_out, vox1_out, vox2_out, vox3_out,
            np_out, vcz_out, vcy_out, vcx_out,
            pbv_vm, pbn_vm, pbc_vm,
            ii0, ii1, ii2, ii3, fi0, fi1, fi2, fi3,
            lf0, lf1, lf2, lf3, li0, li1, li2, li3, sem):
    wid = lax.axis_index("c") * 16 + lax.axis_index("s")
    vox_outs = (vox0_out, vox1_out, vox2_out, vox3_out)
    sp_hbms = (sp0_hbm, sp1_hbm, sp2_hbm, sp3_hbm)
    fins = (fi0, fi1, fi2, fi3)
    locf = (lf0, lf1, lf2, lf3)
    loci = (li0, li1, li2, li3)
    vc_hbms = (vcz_hbm, vcy_hbm, vcx_hbm)
    vc_outs = (vcz_out, vcy_out, vcx_out)
    lane = lax.broadcasted_iota(jnp.int32, (16,), 0)

    pltpu.sync_copy(pbv_hbm, pbv_vm)
    pltpu.sync_copy(pbn_hbm, pbn_vm)
    pltpu.sync_copy(pbc_hbm, pbc_vm)

    zf = jnp.zeros((16,), jnp.float32)
    zi32 = jnp.zeros((16,), jnp.int32)

    @pl.when(wid < VOX_W)
    def _vox():
        def chunk(k, carry):
            c = wid + VOX_W * k

            @pl.when(c < NCHV)
            def _():
                a = c * CH
                p0, p1 = _span(pbv_vm, c)

                def zero(v, cy_):
                    o = v * 16
                    for j in range(4):
                        locf[j][pl.ds(o, 16)] = zf
                    return cy_
                lax.fori_loop(0, CH // 16, zero, 0)

                trips = (p1 - p0 + PIECE_V - 1) // PIECE_V

                def piece(t, cy_):
                    st = p0 + t * PIECE_V
                    hs = [pltpu.async_copy(
                        didxv_hbm.at[pl.ds(st, PIECE_V)],
                        ii0.at[pl.ds(0, PIECE_V)], sem)]
                    for j in range(4):
                        hs.append(pltpu.async_copy(
                            sp_hbms[j].at[pl.ds(st, PIECE_V)], fins[j], sem))
                    for h in hs:
                        h.wait()

                    def vreg(v, cz_):
                        o = v * 16
                        d = ii0[pl.ds(o, 16)]
                        m = (d >= a) & (d < a + CH)
                        li = d - a
                        for j in range(4):
                            x = fins[j][pl.ds(o, 16)]
                            plsc.store_scatter(locf[j], [li], x, mask=m)
                        return cz_
                    lax.fori_loop(0, PIECE_V // 16, vreg, 0)
                    return cy_
                lax.fori_loop(0, trips, piece, 0)

                for j in range(4):
                    pltpu.sync_copy(locf[j], vox_outs[j].at[pl.ds(a, CH)])
            return carry
        lax.fori_loop(0, VOX_ROUNDS, chunk, 0)

    @pl.when((wid >= VOX_W) & (wid < VOX_W + NP_W))
    def _np():
        def chunk(k, carry):
            c = (wid - VOX_W) + NP_W * k

            @pl.when(c < NCHN)
            def _():
                a = c * CH
                p0, p1 = _span(pbn_vm, c)

                def zero(v, cy_):
                    li0[pl.ds(v * 16, 16)] = zi32
                    return cy_
                lax.fori_loop(0, CH // 16, zero, 0)

                trips = (p1 - p0 + PIECE_N - 1) // PIECE_N

                def piece(t, cy_):
                    st = p0 + t * PIECE_N
                    h0 = pltpu.async_copy(
                        didxn_hbm.at[pl.ds(st, PIECE_N)], ii0, sem)
                    h1 = pltpu.async_copy(
                        valn_hbm.at[pl.ds(st, PIECE_N)], ii1, sem)
                    h0.wait()
                    h1.wait()

                    def vreg(v, cz_):
                        o = v * 16
                        d = ii0[pl.ds(o, 16)]
                        m = (d >= a) & (d < a + CH)
                        li = d - a
                        x = ii1[pl.ds(o, 16)]
                        plsc.store_scatter(li0, [li], x, mask=m)
                        return cz_
                    lax.fori_loop(0, PIECE_N // 16, vreg, 0)
                    return cy_
                lax.fori_loop(0, trips, piece, 0)
                pltpu.sync_copy(li0, np_out.at[pl.ds(a, CH)])
            return carry
        lax.fori_loop(0, NP_ROUNDS, chunk, 0)

    @pl.when(wid >= VOX_W + NP_W)
    def _vc():
        def chunk(k, carry):
            c = (wid - VOX_W - NP_W) + VC_W * k

            @pl.when(c < NCHN)
            def _():
                a = c * CH
                p0, p1 = _span(pbc_vm, c)

                def zero(v, cy_):
                    o = v * 16
                    for j in range(3):
                        loci[j][pl.ds(o, 16)] = zi32
                    return cy_
                lax.fori_loop(0, CH // 16, zero, 0)

                trips = (p1 - p0 + PIECE_N - 1) // PIECE_N

                def piece(t, cy_):
                    st = p0 + t * PIECE_N
                    hs = [pltpu.async_copy(
                        didxc_hbm.at[pl.ds(st, PIECE_N)], ii0, sem)]
                    for j in range(3):
                        hs.append(pltpu.async_copy(
                            vc_hbms[j].at[pl.ds(st, PIECE_N)],
                            (ii1, ii2, ii3)[j], sem))
                    for h in hs:
                        h.wait()

                    def vreg(v, cz_):
                        o = v * 16
                        d = ii0[pl.ds(o, 16)]
                        m = (d >= a) & (d < a + CH)
                        li = d - a
                        for j in range(3):
                            x = (ii1, ii2, ii3)[j][pl.ds(o, 16)]
                            plsc.store_scatter(loci[j], [li], x, mask=m)
                        return cz_
                    lax.fori_loop(0, PIECE_N // 16, vreg, 0)
                    return cy_
                lax.fori_loop(0, trips, piece, 0)

                for j in range(3):
                    pltpu.sync_copy(loci[j], vc_outs[j].at[pl.ds(a, CH)])
            return carry
        lax.fori_loop(0, VC_ROUNDS, chunk, 0)


def kernel(points):
    pts_pad = jnp.concatenate(
        [points, jnp.full((PADN, 4), 1e9, jnp.float32)], axis=0)
    keys, invb = _compute_keys(pts_pad)

    iota = jnp.arange(NS, dtype=jnp.int32)
    skey, sidx = lax.sort((keys, iota), num_keys=1, is_stable=True)

    flag = jnp.concatenate([jnp.ones((1,), bool), skey[1:] != skey[:-1]])
    last = jnp.concatenate([skey[:-1] != skey[1:], jnp.ones((1,), bool)])
    uid = jnp.cumsum(flag.astype(jnp.int32)) - 1
    run_start = lax.cummax(jnp.where(flag, iota, 0))
    pos = iota - run_start
    run_len = pos + 1
    valid = skey < SENT
    zs = (skey >= OFF[1]).astype(jnp.int32) + (skey >= OFF[2]).astype(jnp.int32)
    u1 = jnp.sum((flag & (skey < OFF[1])).astype(jnp.int32))
    u2 = jnp.sum((flag & (skey < OFF[2])).astype(jnp.int32))
    ubase = jnp.stack([jnp.zeros((), jnp.int32), u1, u2])[zs]
    sinv = uid - ubase
    nvox_s = jnp.asarray(NVOX, jnp.int32)[zs]
    keep = valid & (pos < MAXP) & (sinv < nvox_s)

    voxb_s = jnp.asarray(VOXB, jnp.int32)[zs]
    didx_v = jnp.where(keep, voxb_s + sinv * MAXP + pos, DUMPV)
    npb_s = jnp.asarray(NPB, jnp.int32)[zs]
    lmask = valid & last & (sinv < nvox_s)
    didx_np = jnp.where(lmask, npb_s + sinv, DUMPN)
    val_np = jnp.minimum(run_len, MAXP)
    fmask = valid & flag & (sinv < nvox_s)
    didx_vc = jnp.where(fmask, npb_s + sinv, DUMPN)
    offv = jnp.asarray(OFF, jnp.int32)[zs]
    gxv = jnp.asarray(GX, jnp.int32)[zs]
    gyv = jnp.asarray(GY, jnp.int32)[zs]
    lk = skey - offv
    cx = lk % gxv
    cyq = lk // gxv
    cy = cyq % gyv
    cz = cyq // gyv

    def search_key(didx, mask, big):
        nxt = lax.rev(lax.cummin(lax.rev(
            jnp.where(mask, didx, big), (0,)), axis=0), (0,))
        return jnp.where(mask, 2 * didx, 2 * nxt - 1).astype(jnp.int32)

    kv = search_key(didx_v, keep, 1_500_000)
    kn = search_key(didx_np, lmask, 500_000)
    kc = search_key(didx_vc, fmask, 500_000)
    bv = jnp.arange(NCHV + 1, dtype=jnp.int32) * (2 * CH)
    bn = jnp.arange(NCHN + 1, dtype=jnp.int32) * (2 * CH)
    pbv = jnp.searchsorted(kv, bv).astype(jnp.int32)
    pbn = jnp.searchsorted(kn, bn).astype(jnp.int32)
    pbc = jnp.searchsorted(kc, bn).astype(jnp.int32)
    pbv = jnp.pad(pbv, (0, PBV_PAD - NCHV - 1))
    pbn = jnp.pad(pbn, (0, PBN_PAD - NCHN - 1))
    pbc = jnp.pad(pbc, (0, PBN_PAD - NCHN - 1))

    spts = pts_pad[sidx]
    pad_i = jnp.full((PIECE_N,), DUMPV, jnp.int32)
    pad_f = jnp.zeros((PIECE_N,), jnp.float32)

    def padi(x):
        return jnp.concatenate([x, pad_i])

    def padf(x):
        return jnp.concatenate([x, pad_f])

    res = _sc_bin(
        padi(didx_v), padf(spts[:, 0]), padf(spts[:, 1]), padf(spts[:, 2]),
        padf(spts[:, 3]),
        padi(didx_np), padi(val_np), padi(didx_vc),
        padi(cz), padi(cy), padi(cx),
        pbv, pbn, pbc)
    vox_planes = res[:4]
    npts_flat = res[4]
    vc_planes = res[5:8]

    iota_n = iota[:N]
    fills = []
    for zi in range(3):
        bit = (invb[:N] >> zi) & 1
        istar = jnp.max(jnp.where(bit == 1, iota_n, -1))
        istar = jnp.maximum(istar, 0)
        p = points[istar]
        fx = jnp.floor((p[0] + np.float32(100.0)) / np.float32(VSX[zi])).astype(jnp.int32)
        fy = jnp.floor((p[1] + np.float32(100.0)) / np.float32(VSX[zi])).astype(jnp.int32)
        fz = jnp.floor((p[2] + np.float32(5.0)) / np.float32(VSZ[zi])).astype(jnp.int32)
        fills.append(jnp.stack([fz, fy, fx]))

    vox_flat = jnp.stack(vox_planes, axis=1)
    vc_flat = jnp.stack(vc_planes, axis=1)
    outs = []
    for zi in range(3):
        v = vox_flat[VOXB[zi]:VOXB[zi] + NVOX[zi] * MAXP].reshape(
            NVOX[zi], MAXP, 4)
        npts = npts_flat[NPB[zi]:NPB[zi] + NVOX[zi]]
        vc = vc_flat[NPB[zi]:NPB[zi] + NVOX[zi]]
        vc = jnp.where((npts > 0)[:, None], vc, fills[zi][None, :])
        outs.extend([v, vc, npts])
    return tuple(outs)
